# Initial kernel scaffold; baseline (speedup 1.0000x reference)
import jax
import jax.numpy as jnp
from jax.experimental import pallas as pl


def kernel(x, edge_index, etype, V1, comb1, Wself1, b1, V2, comb2, Wself2, b2, Wagg, bagg, Wd1, bd1, Wd2, bd2, Wd3, bd3):
    raise NotImplementedError("write your pallas kernel here")


# speedup vs baseline: 11.6819x; 11.6819x over previous
import functools

import jax
import jax.numpy as jnp
from jax import lax
from jax.experimental import pallas as pl
from jax.experimental.pallas import tpu as pltpu
from jax.experimental.pallas import tpu_sc as plsc

N = 10000
E = 320000
F = 128
R = 8
NB = 8

NC = 2
NS = 16
NW = NC * NS

CH = 128
NCHUNK = 80
EPT = CH * NCHUNK
EPAD = EPT * NW
NPAD = 10240
RPT = NPAD // NS

BLK = 400
GRID = N // BLK


def _proj_body(first, *refs):
    if first:
        x_ref, V_ref, comb_ref, Wself_ref, b_ref, htab_ref, self_ref, wcat = refs
        xb = x_ref[...]
    else:
        p0_ref, p1_ref, s_ref, V_ref, comb_ref, Wself_ref, b_ref, \
            htab_ref, self_ref, wcat = refs
        xb = p0_ref[...] + p1_ref[...] + s_ref[...]
        xb = jnp.where(xb > 0, xb, 0.01 * xb)

    i = pl.program_id(0)
    r = pl.program_id(1)

    @pl.when((i == 0) & (r == 0))
    def _build():
        for rr in range(R):
            acc = comb_ref[rr, 0] * V_ref[0]
            for b in range(1, NB):
                acc = acc + comb_ref[rr, b] * V_ref[b]
            wcat[:, rr * F:(rr + 1) * F] = acc

    htab_ref[0] = jnp.dot(xb, wcat[pl.ds(0, F), pl.ds(r * F, F)],
                          precision=lax.Precision.HIGHEST,
                          preferred_element_type=jnp.float32)

    @pl.when(r == 0)
    def _self():
        self_ref[...] = (
            jnp.dot(xb, Wself_ref[...], precision=lax.Precision.HIGHEST,
                    preferred_element_type=jnp.float32)
            + b_ref[...]
        )


def _make_proj(first):
    node_in = pl.BlockSpec((BLK, F), lambda i, r: (i, 0))
    in_specs = ([node_in] if first else [node_in, node_in, node_in]) + [
        pl.BlockSpec((NB, F, F), lambda i, r: (0, 0, 0)),
        pl.BlockSpec((R, NB), lambda i, r: (0, 0), memory_space=pltpu.SMEM),
        pl.BlockSpec((F, F), lambda i, r: (0, 0)),
        pl.BlockSpec((1, F), lambda i, r: (0, 0)),
    ]
    return pl.pallas_call(
        functools.partial(_proj_body, first),
        grid=(GRID, R),
        in_specs=in_specs,
        out_specs=[
            pl.BlockSpec((1, BLK, F), lambda i, r: (r, i, 0)),
            pl.BlockSpec((BLK, F), lambda i, r: (i, 0)),
        ],
        out_shape=[
            jax.ShapeDtypeStruct((R, N, F), jnp.float32),
            jax.ShapeDtypeStruct((N, F), jnp.float32),
        ],
        scratch_shapes=[pltpu.VMEM((F, R * F), jnp.float32)],
    )


_proj_first = _make_proj(True)
_proj_mid = _make_proj(False)


def _edge_agg_body(htab, src2, et2, dst3, out, srcv, etv, dstv, rows, aggsh,
                   sem):
    cid = lax.axis_index("c")
    sid = lax.axis_index("s")
    wid = cid * NS + sid

    pltpu.sync_copy(src2.at[wid], srcv)
    pltpu.sync_copy(et2.at[wid], etv)
    pltpu.sync_copy(dst3.at[wid], dstv)

    def _zrow(i, c):
        for q in range(F // 16):
            rows[i, pl.ds(q * 16, 16)] = jnp.zeros((16,), jnp.float32)
        return c
    lax.fori_loop(0, CH, _zrow, 0)
    for k in range(RPT // CH):
        pltpu.sync_copy(rows, aggsh.at[pl.ds(sid * RPT + k * CH, CH)])

    def _gidx(i, c):
        off = pl.multiple_of(i * 16, 16)
        srcv[pl.ds(off, 16)] = etv[pl.ds(off, 16)] * N + srcv[pl.ds(off, 16)]
        return c
    lax.fori_loop(0, EPT // 16, _gidx, 0)

    plsc.subcore_barrier()

    def _chunk(j, c):
        off = pl.multiple_of(j * CH, CH)
        pltpu.async_copy(htab.at[srcv.at[pl.ds(off, CH)]], rows, sem).wait()
        pltpu.sync_copy(rows, aggsh.at[dstv.at[j]], add=True)
        return c
    lax.fori_loop(0, NCHUNK, _chunk, 0)

    plsc.subcore_barrier()

    pltpu.sync_copy(aggsh.at[pl.ds(sid * RPT, RPT)],
                    out.at[cid, pl.ds(sid * RPT, RPT)])


@functools.cache
def _get_edge_agg():
    mesh = plsc.VectorSubcoreMesh(
        core_axis_name="c", subcore_axis_name="s",
        num_cores=NC, num_subcores=NS)
    return pl.kernel(
        _edge_agg_body,
        out_type=jax.ShapeDtypeStruct((NC, NPAD, F), jnp.float32),
        mesh=mesh,
        scratch_types=[
            pltpu.VMEM((EPT,), jnp.int32),
            pltpu.VMEM((EPT,), jnp.int32),
            pltpu.VMEM((NCHUNK, CH), jnp.int32),
            pltpu.VMEM((CH, F), jnp.float32),
            pltpu.VMEM_SHARED((NPAD, F), jnp.float32),
            pltpu.SemaphoreType.DMA,
        ],
    )


def _edge_agg(htab, src2, et2, dst3):
    return _get_edge_agg()(htab, src2, et2, dst3)


def _head_body(p0_ref, p1_ref, s_ref, Wagg_ref, bagg_ref, Wd1_ref, bd1_ref,
               Wd2_ref, bd2_ref, Wd3_ref, bd3_ref, out_ref, accG, accS):
    i = pl.program_id(0)
    h2 = p0_ref[...] + p1_ref[...] + s_ref[...]
    wd1 = Wd1_ref[...]
    g = lax.dot_general(h2, wd1, (((0,), (0,)), ((), ())),
                        precision=lax.Precision.HIGHEST,
                        preferred_element_type=jnp.float32)

    @pl.when(i == 0)
    def _init():
        accG[...] = g
        accS[...] = jnp.zeros((8, F), jnp.float32)

    @pl.when(i > 0)
    def _acc():
        accG[...] = accG[...] + g

    accS[0:1, 0:100] = accS[0:1, 0:100] + jnp.sum(wd1, axis=0, keepdims=True)

    @pl.when(i == GRID - 1)
    def _final():
        u = lax.dot_general(Wagg_ref[...], accG[...],
                            (((0,), (0,)), ((), ())),
                            precision=lax.Precision.HIGHEST,
                            preferred_element_type=jnp.float32)
        u = u + bagg_ref[...] * accS[0:1, 0:100] + bd1_ref[...]
        t = jnp.dot(u, Wd2_ref[...], precision=lax.Precision.HIGHEST,
                    preferred_element_type=jnp.float32)
        t = t + bd2_ref[...]
        t = jnp.where(t > 0, t, 0.01 * t)
        out_ref[...] = (
            jnp.dot(t, Wd3_ref[...], precision=lax.Precision.HIGHEST,
                    preferred_element_type=jnp.float32)
            + bd3_ref[...]
        )


_head = pl.pallas_call(
    _head_body,
    grid=(GRID,),
    in_specs=[
        pl.BlockSpec((BLK, F), lambda i: (i, 0)),
        pl.BlockSpec((BLK, F), lambda i: (i, 0)),
        pl.BlockSpec((BLK, F), lambda i: (i, 0)),
        pl.BlockSpec((F, 1), lambda i: (0, 0)),
        pl.BlockSpec((1, 1), lambda i: (0, 0)),
        pl.BlockSpec((BLK, 100), lambda i: (i, 0)),
        pl.BlockSpec((1, 100), lambda i: (0, 0)),
        pl.BlockSpec((100, 20), lambda i: (0, 0)),
        pl.BlockSpec((1, 20), lambda i: (0, 0)),
        pl.BlockSpec((20, 10), lambda i: (0, 0)),
        pl.BlockSpec((1, 10), lambda i: (0, 0)),
    ],
    out_specs=pl.BlockSpec((1, 10), lambda i: (0, 0)),
    out_shape=jax.ShapeDtypeStruct((1, 10), jnp.float32),
    scratch_shapes=[
        pltpu.VMEM((F, 100), jnp.float32),
        pltpu.VMEM((8, F), jnp.float32),
    ],
)


def kernel(x, edge_index, etype, V1, comb1, Wself1, b1, V2, comb2, Wself2, b2,
           Wagg, bagg, Wd1, bd1, Wd2, bd2, Wd3, bd3):
    src = edge_index[0].astype(jnp.int32)
    dst = edge_index[1].astype(jnp.int32)
    et = etype.astype(jnp.int32)

    pad = EPAD - E
    src2 = jnp.concatenate([src, jnp.zeros((pad,), jnp.int32)]).reshape(NW, EPT)
    et2 = jnp.concatenate([et, jnp.zeros((pad,), jnp.int32)]).reshape(NW, EPT)
    dst3 = jnp.concatenate([dst, jnp.full((pad,), N, jnp.int32)]).reshape(
        NW, NCHUNK, CH)

    htab1, self1 = _proj_first(x, V1, comb1, Wself1, b1.reshape(1, F))
    parts1 = _edge_agg(htab1.reshape(R * N, F), src2, et2, dst3)

    htab2, self2 = _proj_mid(parts1[0, :N], parts1[1, :N], self1,
                             V2, comb2, Wself2, b2.reshape(1, F))
    parts2 = _edge_agg(htab2.reshape(R * N, F), src2, et2, dst3)

    return _head(parts2[0, :N], parts2[1, :N], self2,
                 Wagg, bagg.reshape(1, 1),
                 Wd1, bd1.reshape(1, 100),
                 Wd2, bd2.reshape(1, 20),
                 Wd3, bd3.reshape(1, 10))


# speedup vs baseline: 14.8782x; 1.2736x over previous
import functools

import jax
import jax.numpy as jnp
from jax import lax
from jax.experimental import pallas as pl
from jax.experimental.pallas import tpu as pltpu
from jax.experimental.pallas import tpu_sc as plsc

N = 10000
E = 320000
F = 128
R = 8
NB = 8

NC = 2
NS = 16
NW = NC * NS

CH = 128
NCHUNK = 80
NPH = 2
NCH_P = NCHUNK // NPH
EPP = CH * NCH_P
EPT = CH * NCHUNK
EPAD = EPT * NW
NPAD = 10240
RPT = NPAD // NS

BLK = 400
GRID = N // BLK


def _proj_body(first, *refs):
    if first:
        x_ref, V_ref, comb_ref, Wself_ref, b_ref, htab_ref, self_ref = refs
        xb = x_ref[...]
    else:
        p0_ref, p1_ref, s_ref, V_ref, comb_ref, Wself_ref, b_ref, \
            htab_ref, self_ref = refs
        xb = p0_ref[...] + p1_ref[...] + s_ref[...]
        xb = jnp.where(xb > 0, xb, 0.01 * xb)

    hbs = [jnp.dot(xb, V_ref[b], preferred_element_type=jnp.float32)
           for b in range(NB)]
    for r in range(R):
        acc = comb_ref[r, 0] * hbs[0]
        for b in range(1, NB):
            acc = acc + comb_ref[r, b] * hbs[b]
        htab_ref[r] = acc

    self_ref[...] = (
        jnp.dot(xb, Wself_ref[...], preferred_element_type=jnp.float32)
        + b_ref[...]
    )


def _make_proj(first):
    node_in = pl.BlockSpec((BLK, F), lambda i: (i, 0))
    in_specs = ([node_in] if first else [node_in, node_in, node_in]) + [
        pl.BlockSpec((NB, F, F), lambda i: (0, 0, 0)),
        pl.BlockSpec((R, NB), lambda i: (0, 0), memory_space=pltpu.SMEM),
        pl.BlockSpec((F, F), lambda i: (0, 0)),
        pl.BlockSpec((1, F), lambda i: (0, 0)),
    ]
    return pl.pallas_call(
        functools.partial(_proj_body, first),
        grid=(GRID,),
        in_specs=in_specs,
        out_specs=[
            pl.BlockSpec((R, BLK, F), lambda i: (0, i, 0)),
            pl.BlockSpec((BLK, F), lambda i: (i, 0)),
        ],
        out_shape=[
            jax.ShapeDtypeStruct((R, N, F), jnp.float32),
            jax.ShapeDtypeStruct((N, F), jnp.float32),
        ],
    )


_proj_first = _make_proj(True)
_proj_mid = _make_proj(False)


def _edge_agg_body(htab, src2, et2, dst3, out, srcv, etv, dstv, rowsA, rowsB,
                   aggsh, semA, semB):
    cid = lax.axis_index("c")
    sid = lax.axis_index("s")
    wid = cid * NS + sid

    def _zrow(i, c):
        for q in range(F // 16):
            rowsA[i, pl.ds(q * 16, 16)] = jnp.zeros((16,), jnp.float32)
        return c
    lax.fori_loop(0, CH, _zrow, 0)
    for k in range(RPT // CH):
        pltpu.sync_copy(rowsA, aggsh.at[pl.ds(sid * RPT + k * CH, CH)])

    plsc.subcore_barrier()

    for p in range(NPH):
        pltpu.sync_copy(src2.at[wid, pl.ds(p * EPP, EPP)], srcv)
        pltpu.sync_copy(et2.at[wid, pl.ds(p * EPP, EPP)], etv)
        pltpu.sync_copy(dst3.at[wid, pl.ds(p * NCH_P, NCH_P)], dstv)

        def _gidx(i, c):
            off = pl.multiple_of(i * 16, 16)
            srcv[pl.ds(off, 16)] = (
                etv[pl.ds(off, 16)] * N + srcv[pl.ds(off, 16)])
            return c
        lax.fori_loop(0, EPP // 16, _gidx, 0)

        pltpu.async_copy(htab.at[srcv.at[pl.ds(0, CH)]], rowsA, semA)

        def _pair(jj, c):
            off0 = pl.multiple_of(jj * 2 * CH, CH)
            off1 = pl.multiple_of(jj * 2 * CH + CH, CH)
            pltpu.make_async_copy(
                htab.at[srcv.at[pl.ds(off0, CH)]], rowsA, semA).wait()
            pltpu.async_copy(htab.at[srcv.at[pl.ds(off1, CH)]], rowsB, semB)
            pltpu.sync_copy(rowsA, aggsh.at[dstv.at[jj * 2]], add=True)
            pltpu.make_async_copy(
                htab.at[srcv.at[pl.ds(off1, CH)]], rowsB, semB).wait()

            @pl.when(jj < NCH_P // 2 - 1)
            def _refill():
                off2 = pl.multiple_of(jj * 2 * CH + 2 * CH, CH)
                pltpu.async_copy(
                    htab.at[srcv.at[pl.ds(off2, CH)]], rowsA, semA)

            pltpu.sync_copy(rowsB, aggsh.at[dstv.at[jj * 2 + 1]], add=True)
            return c
        lax.fori_loop(0, NCH_P // 2, _pair, 0)

    plsc.subcore_barrier()

    pltpu.sync_copy(aggsh.at[pl.ds(sid * RPT, RPT)],
                    out.at[cid, pl.ds(sid * RPT, RPT)])


@functools.cache
def _get_edge_agg():
    mesh = plsc.VectorSubcoreMesh(
        core_axis_name="c", subcore_axis_name="s",
        num_cores=NC, num_subcores=NS)
    return pl.kernel(
        _edge_agg_body,
        out_type=jax.ShapeDtypeStruct((NC, NPAD, F), jnp.float32),
        mesh=mesh,
        scratch_types=[
            pltpu.VMEM((EPP,), jnp.int32),
            pltpu.VMEM((EPP,), jnp.int32),
            pltpu.VMEM((NCH_P, CH), jnp.int32),
            pltpu.VMEM((CH, F), jnp.float32),
            pltpu.VMEM((CH, F), jnp.float32),
            pltpu.VMEM_SHARED((NPAD, F), jnp.float32),
            pltpu.SemaphoreType.DMA,
            pltpu.SemaphoreType.DMA,
        ],
    )


def _edge_agg(htab, src2, et2, dst3):
    return _get_edge_agg()(htab, src2, et2, dst3)


def _head_body(p0_ref, p1_ref, s_ref, Wagg_ref, bagg_ref, Wd1_ref, bd1_ref,
               Wd2_ref, bd2_ref, Wd3_ref, bd3_ref, out_ref, accU):
    i = pl.program_id(0)
    h2 = p0_ref[...] + p1_ref[...] + s_ref[...]
    z = jnp.dot(h2, Wagg_ref[...], preferred_element_type=jnp.float32)
    z = z + bagg_ref[...]
    c = lax.dot_general(z, Wd1_ref[...], (((0,), (0,)), ((), ())),
                        preferred_element_type=jnp.float32)

    @pl.when(i == 0)
    def _init():
        accU[...] = jnp.zeros((8, F), jnp.float32)

    accU[0:1, 0:100] = accU[0:1, 0:100] + c

    @pl.when(i == GRID - 1)
    def _final():
        u = accU[0:1, 0:100] + bd1_ref[...]
        t = jnp.dot(u, Wd2_ref[...], preferred_element_type=jnp.float32)
        t = t + bd2_ref[...]
        t = jnp.where(t > 0, t, 0.01 * t)
        out_ref[...] = (
            jnp.dot(t, Wd3_ref[...], preferred_element_type=jnp.float32)
            + bd3_ref[...]
        )


_head = pl.pallas_call(
    _head_body,
    grid=(GRID,),
    in_specs=[
        pl.BlockSpec((BLK, F), lambda i: (i, 0)),
        pl.BlockSpec((BLK, F), lambda i: (i, 0)),
        pl.BlockSpec((BLK, F), lambda i: (i, 0)),
        pl.BlockSpec((F, 1), lambda i: (0, 0)),
        pl.BlockSpec((1, 1), lambda i: (0, 0)),
        pl.BlockSpec((BLK, 100), lambda i: (i, 0)),
        pl.BlockSpec((1, 100), lambda i: (0, 0)),
        pl.BlockSpec((100, 20), lambda i: (0, 0)),
        pl.BlockSpec((1, 20), lambda i: (0, 0)),
        pl.BlockSpec((20, 10), lambda i: (0, 0)),
        pl.BlockSpec((1, 10), lambda i: (0, 0)),
    ],
    out_specs=pl.BlockSpec((1, 10), lambda i: (0, 0)),
    out_shape=jax.ShapeDtypeStruct((1, 10), jnp.float32),
    scratch_shapes=[
        pltpu.VMEM((8, F), jnp.float32),
    ],
)


def kernel(x, edge_index, etype, V1, comb1, Wself1, b1, V2, comb2, Wself2, b2,
           Wagg, bagg, Wd1, bd1, Wd2, bd2, Wd3, bd3):
    src = edge_index[0].astype(jnp.int32)
    dst = edge_index[1].astype(jnp.int32)
    et = etype.astype(jnp.int32)

    pad = EPAD - E
    src2 = jnp.concatenate([src, jnp.zeros((pad,), jnp.int32)]).reshape(NW, EPT)
    et2 = jnp.concatenate([et, jnp.zeros((pad,), jnp.int32)]).reshape(NW, EPT)
    dst3 = jnp.concatenate([dst, jnp.full((pad,), N, jnp.int32)]).reshape(
        NW, NCHUNK, CH)

    htab1, self1 = _proj_first(x, V1, comb1, Wself1, b1.reshape(1, F))
    parts1 = _edge_agg(htab1.reshape(R * N, F), src2, et2, dst3)

    htab2, self2 = _proj_mid(parts1[0, :N], parts1[1, :N], self1,
                             V2, comb2, Wself2, b2.reshape(1, F))
    parts2 = _edge_agg(htab2.reshape(R * N, F), src2, et2, dst3)

    return _head(parts2[0, :N], parts2[1, :N], self2,
                 Wagg, bagg.reshape(1, 1),
                 Wd1, bd1.reshape(1, 100),
                 Wd2, bd2.reshape(1, 20),
                 Wd3, bd3.reshape(1, 10))


# speedup vs baseline: 14.8921x; 1.0009x over previous
import functools

import jax
import jax.numpy as jnp
from jax import lax
from jax.experimental import pallas as pl
from jax.experimental.pallas import tpu as pltpu
from jax.experimental.pallas import tpu_sc as plsc

N = 10000
E = 320000
F = 128
R = 8
NB = 8

NC = 2
NS = 16
NW = NC * NS

CH = 128
NCHUNK = 80
NPH = 2
NCH_P = NCHUNK // NPH
EPP = CH * NCH_P
EPT = CH * NCHUNK
EPAD = EPT * NW
NPAD = 10240
RPT = NPAD // NS

BLK = 400
GRID = N // BLK


def _proj_body(first, *refs):
    if first:
        x_ref, V_ref, comb_ref, Wself_ref, b_ref, htab_ref, self_ref = refs
        xb = x_ref[...]
    else:
        p0_ref, p1_ref, s_ref, V_ref, comb_ref, Wself_ref, b_ref, \
            htab_ref, self_ref = refs
        xb = p0_ref[...] + p1_ref[...] + s_ref[...]
        xb = jnp.where(xb > 0, xb, 0.01 * xb)

    hbs = [jnp.dot(xb, V_ref[b], preferred_element_type=jnp.float32)
           for b in range(NB)]
    for r in range(R):
        acc = comb_ref[r, 0] * hbs[0]
        for b in range(1, NB):
            acc = acc + comb_ref[r, b] * hbs[b]
        htab_ref[r] = acc

    self_ref[...] = (
        jnp.dot(xb, Wself_ref[...], preferred_element_type=jnp.float32)
        + b_ref[...]
    )


def _make_proj(first):
    node_in = pl.BlockSpec((BLK, F), lambda i: (i, 0))
    in_specs = ([node_in] if first else [node_in, node_in, node_in]) + [
        pl.BlockSpec((NB, F, F), lambda i: (0, 0, 0)),
        pl.BlockSpec((R, NB), lambda i: (0, 0), memory_space=pltpu.SMEM),
        pl.BlockSpec((F, F), lambda i: (0, 0)),
        pl.BlockSpec((1, F), lambda i: (0, 0)),
    ]
    return pl.pallas_call(
        functools.partial(_proj_body, first),
        grid=(GRID,),
        in_specs=in_specs,
        out_specs=[
            pl.BlockSpec((R, BLK, F), lambda i: (0, i, 0)),
            pl.BlockSpec((BLK, F), lambda i: (i, 0)),
        ],
        out_shape=[
            jax.ShapeDtypeStruct((R, N, F), jnp.float32),
            jax.ShapeDtypeStruct((N, F), jnp.float32),
        ],
    )


_proj_first = _make_proj(True)
_proj_mid = _make_proj(False)


def _edge_agg_body(htab, src2, et2, dst3, out, srcv, etv, dstv, rowsA, rowsB,
                   aggsh, semA, semB):
    cid = lax.axis_index("c")
    sid = lax.axis_index("s")
    wid = cid * NS + sid

    def _zrow(i, c):
        for q in range(F // 16):
            rowsA[i, pl.ds(q * 16, 16)] = jnp.zeros((16,), jnp.float32)
        return c
    lax.fori_loop(0, CH, _zrow, 0)
    for k in range(RPT // CH):
        pltpu.sync_copy(rowsA, aggsh.at[pl.ds(sid * RPT + k * CH, CH)])

    plsc.subcore_barrier()

    for p in range(NPH):
        pltpu.sync_copy(src2.at[wid, pl.ds(p * EPP, EPP)], srcv)
        pltpu.sync_copy(et2.at[wid, pl.ds(p * EPP, EPP)], etv)
        pltpu.sync_copy(dst3.at[wid, pl.ds(p * NCH_P, NCH_P)], dstv)

        def _gidx(i, c):
            off = pl.multiple_of(i * 16, 16)
            srcv[pl.ds(off, 16)] = (
                etv[pl.ds(off, 16)] * N + srcv[pl.ds(off, 16)])
            return c
        lax.fori_loop(0, EPP // 16, _gidx, 0)

        pltpu.async_copy(htab.at[srcv.at[pl.ds(0, CH)]], rowsA, semA)

        def _pair(jj, c):
            off0 = pl.multiple_of(jj * 2 * CH, CH)
            off1 = pl.multiple_of(jj * 2 * CH + CH, CH)
            pltpu.make_async_copy(
                htab.at[srcv.at[pl.ds(off0, CH)]], rowsA, semA).wait()
            pltpu.async_copy(htab.at[srcv.at[pl.ds(off1, CH)]], rowsB, semB)
            pltpu.sync_copy(rowsA, aggsh.at[dstv.at[jj * 2]], add=True)
            pltpu.make_async_copy(
                htab.at[srcv.at[pl.ds(off1, CH)]], rowsB, semB).wait()

            @pl.when(jj < NCH_P // 2 - 1)
            def _refill():
                off2 = pl.multiple_of(jj * 2 * CH + 2 * CH, CH)
                pltpu.async_copy(
                    htab.at[srcv.at[pl.ds(off2, CH)]], rowsA, semA)

            pltpu.sync_copy(rowsB, aggsh.at[dstv.at[jj * 2 + 1]], add=True)
            return c
        lax.fori_loop(0, NCH_P // 2, _pair, 0)

    plsc.subcore_barrier()

    pltpu.sync_copy(aggsh.at[pl.ds(sid * RPT, RPT)],
                    out.at[cid, pl.ds(sid * RPT, RPT)])


@functools.cache
def _get_edge_agg():
    mesh = plsc.VectorSubcoreMesh(
        core_axis_name="c", subcore_axis_name="s",
        num_cores=NC, num_subcores=NS)
    return pl.kernel(
        _edge_agg_body,
        out_type=jax.ShapeDtypeStruct((NC, NPAD, F), jnp.float32),
        mesh=mesh,
        scratch_types=[
            pltpu.VMEM((EPP,), jnp.int32),
            pltpu.VMEM((EPP,), jnp.int32),
            pltpu.VMEM((NCH_P, CH), jnp.int32),
            pltpu.VMEM((CH, F), jnp.float32),
            pltpu.VMEM((CH, F), jnp.float32),
            pltpu.VMEM_SHARED((NPAD, F), jnp.float32),
            pltpu.SemaphoreType.DMA,
            pltpu.SemaphoreType.DMA,
        ],
    )


def _edge_agg(htab, src2, et2, dst3):
    return _get_edge_agg()(htab, src2, et2, dst3)


def _head_body(p0_ref, p1_ref, s_ref, Wagg_ref, bagg_ref, Wd1_ref, bd1_ref,
               Wd2_ref, bd2_ref, Wd3_ref, bd3_ref, out_ref, accU):
    i = pl.program_id(0)
    h2 = p0_ref[...] + p1_ref[...] + s_ref[...]
    z = jnp.dot(h2, Wagg_ref[...], preferred_element_type=jnp.float32)
    z = z + bagg_ref[...]
    c = lax.dot_general(z, Wd1_ref[...], (((0,), (0,)), ((), ())),
                        preferred_element_type=jnp.float32)

    @pl.when(i == 0)
    def _init():
        accU[...] = jnp.zeros((8, F), jnp.float32)

    accU[0:1, 0:100] = accU[0:1, 0:100] + c

    @pl.when(i == GRID - 1)
    def _final():
        u = accU[0:1, 0:100] + bd1_ref[...]
        t = jnp.dot(u, Wd2_ref[...], preferred_element_type=jnp.float32)
        t = t + bd2_ref[...]
        t = jnp.where(t > 0, t, 0.01 * t)
        out_ref[...] = (
            jnp.dot(t, Wd3_ref[...], preferred_element_type=jnp.float32)
            + bd3_ref[...]
        )


_head = pl.pallas_call(
    _head_body,
    grid=(GRID,),
    in_specs=[
        pl.BlockSpec((BLK, F), lambda i: (i, 0)),
        pl.BlockSpec((BLK, F), lambda i: (i, 0)),
        pl.BlockSpec((BLK, F), lambda i: (i, 0)),
        pl.BlockSpec((F, 1), lambda i: (0, 0)),
        pl.BlockSpec((1, 1), lambda i: (0, 0)),
        pl.BlockSpec((BLK, 100), lambda i: (i, 0)),
        pl.BlockSpec((1, 100), lambda i: (0, 0)),
        pl.BlockSpec((100, 20), lambda i: (0, 0)),
        pl.BlockSpec((1, 20), lambda i: (0, 0)),
        pl.BlockSpec((20, 10), lambda i: (0, 0)),
        pl.BlockSpec((1, 10), lambda i: (0, 0)),
    ],
    out_specs=pl.BlockSpec((1, 10), lambda i: (0, 0)),
    out_shape=jax.ShapeDtypeStruct((1, 10), jnp.float32),
    scratch_shapes=[
        pltpu.VMEM((8, F), jnp.float32),
    ],
)


def kernel(x, edge_index, etype, V1, comb1, Wself1, b1, V2, comb2, Wself2, b2,
           Wagg, bagg, Wd1, bd1, Wd2, bd2, Wd3, bd3):
    src = edge_index[0].astype(jnp.int32)
    dst = edge_index[1].astype(jnp.int32)
    et = etype.astype(jnp.int32)

    pad = EPAD - E
    src2 = jnp.concatenate([src, jnp.zeros((pad,), jnp.int32)]).reshape(NW, EPT)
    et2 = jnp.concatenate([et, jnp.zeros((pad,), jnp.int32)]).reshape(NW, EPT)
    trash = N + jnp.arange(pad, dtype=jnp.int32) % (NPAD - N)
    dst3 = jnp.concatenate([dst, trash]).reshape(NW, NCHUNK, CH)

    htab1, self1 = _proj_first(x, V1, comb1, Wself1, b1.reshape(1, F))
    parts1 = _edge_agg(htab1.reshape(R * N, F), src2, et2, dst3)

    htab2, self2 = _proj_mid(parts1[0, :N], parts1[1, :N], self1,
                             V2, comb2, Wself2, b2.reshape(1, F))
    parts2 = _edge_agg(htab2.reshape(R * N, F), src2, et2, dst3)

    return _head(parts2[0, :N], parts2[1, :N], self2,
                 Wagg, bagg.reshape(1, 1),
                 Wd1, bd1.reshape(1, 100),
                 Wd2, bd2.reshape(1, 20),
                 Wd3, bd3.reshape(1, 10))


# speedup vs baseline: 16.5402x; 1.1107x over previous
import functools

import jax
import jax.numpy as jnp
from jax import lax
from jax.experimental import pallas as pl
from jax.experimental.pallas import tpu as pltpu
from jax.experimental.pallas import tpu_sc as plsc

N = 10000
E = 320000
F = 128
R = 8
NB = 8

NC = 2
NS = 16
NW = NC * NS

CH = 64
NCHUNK = 160
NPH = 4
NCH_P = NCHUNK // NPH
EPP = CH * NCH_P
EPT = CH * NCHUNK
EPAD = EPT * NW
NBUF = 4
NPAD = 10240
RPT = NPAD // NS

BLK = 400
GRID = N // BLK


def _proj_body(first, *refs):
    if first:
        x_ref, V_ref, comb_ref, Wself_ref, b_ref, htab_ref, self_ref = refs
        xb = x_ref[...]
    else:
        p0_ref, p1_ref, s_ref, V_ref, comb_ref, Wself_ref, b_ref, \
            htab_ref, self_ref = refs
        xb = p0_ref[...] + p1_ref[...] + s_ref[...]
        xb = jnp.where(xb > 0, xb, 0.01 * xb)

    hbs = [jnp.dot(xb, V_ref[b], preferred_element_type=jnp.float32)
           for b in range(NB)]
    for r in range(R):
        acc = comb_ref[r, 0] * hbs[0]
        for b in range(1, NB):
            acc = acc + comb_ref[r, b] * hbs[b]
        htab_ref[r] = acc

    self_ref[...] = (
        jnp.dot(xb, Wself_ref[...], preferred_element_type=jnp.float32)
        + b_ref[...]
    )


def _make_proj(first):
    node_in = pl.BlockSpec((BLK, F), lambda i: (i, 0))
    in_specs = ([node_in] if first else [node_in, node_in, node_in]) + [
        pl.BlockSpec((NB, F, F), lambda i: (0, 0, 0)),
        pl.BlockSpec((R, NB), lambda i: (0, 0), memory_space=pltpu.SMEM),
        pl.BlockSpec((F, F), lambda i: (0, 0)),
        pl.BlockSpec((1, F), lambda i: (0, 0)),
    ]
    return pl.pallas_call(
        functools.partial(_proj_body, first),
        grid=(GRID,),
        in_specs=in_specs,
        out_specs=[
            pl.BlockSpec((R, BLK, F), lambda i: (0, i, 0)),
            pl.BlockSpec((BLK, F), lambda i: (i, 0)),
        ],
        out_shape=[
            jax.ShapeDtypeStruct((R, N, F), jnp.float32),
            jax.ShapeDtypeStruct((N, F), jnp.float32),
        ],
    )


_proj_first = _make_proj(True)
_proj_mid = _make_proj(False)


def _edge_agg_body(htab, src2, et2, dst3, out, srcv, etv, dstv,
                   r0, r1, r2, r3, aggsh, g0, g1, g2, g3, ssem):
    rows = (r0, r1, r2, r3)
    gsems = (g0, g1, g2, g3)
    cid = lax.axis_index("c")
    sid = lax.axis_index("s")
    wid = cid * NS + sid

    def _zrow(i, c):
        for q in range(F // 16):
            r0[i, pl.ds(q * 16, 16)] = jnp.zeros((16,), jnp.float32)
        return c
    lax.fori_loop(0, CH, _zrow, 0)
    for k in range(RPT // CH):
        pltpu.sync_copy(r0, aggsh.at[pl.ds(sid * RPT + k * CH, CH)])

    plsc.subcore_barrier()

    for p in range(NPH):
        pltpu.sync_copy(src2.at[wid, pl.ds(p * EPP, EPP)], srcv)
        pltpu.sync_copy(et2.at[wid, pl.ds(p * EPP, EPP)], etv)
        pltpu.sync_copy(dst3.at[wid, pl.ds(p * NCH_P, NCH_P)], dstv)

        def _gidx(i, c):
            off = pl.multiple_of(i * 16, 16)
            srcv[pl.ds(off, 16)] = (
                etv[pl.ds(off, 16)] * N + srcv[pl.ds(off, 16)])
            return c
        lax.fori_loop(0, EPP // 16, _gidx, 0)

        for q in range(NBUF - 1):
            pltpu.async_copy(
                htab.at[srcv.at[pl.ds(q * CH, CH)]], rows[q], gsems[q])

        def _ring(jj, c):
            for q in range(NBUF):
                j = jj * NBUF + q
                off = pl.multiple_of(j * CH, CH)
                pltpu.make_async_copy(
                    htab.at[srcv.at[pl.ds(off, CH)]], rows[q],
                    gsems[q]).wait()
                pltpu.async_copy(rows[q], aggsh.at[dstv.at[j]], ssem,
                                 add=True)
                if p == 0 and q == 0:
                    @pl.when(jj > 0)
                    def _drain0():
                        pltpu.make_async_copy(
                            rows[q], aggsh.at[dstv.at[j]], ssem).wait()
                else:
                    pltpu.make_async_copy(
                        rows[q], aggsh.at[dstv.at[j]], ssem).wait()

                @pl.when(j + NBUF - 1 < NCH_P)
                def _refill():
                    offn = pl.multiple_of((j + NBUF - 1) * CH, CH)
                    pltpu.async_copy(
                        htab.at[srcv.at[pl.ds(offn, CH)]],
                        rows[(q + NBUF - 1) % NBUF],
                        gsems[(q + NBUF - 1) % NBUF])
            return c
        lax.fori_loop(0, NCH_P // NBUF, _ring, 0)

    pltpu.make_async_copy(r0, aggsh.at[dstv.at[0]], ssem).wait()

    plsc.subcore_barrier()

    pltpu.sync_copy(aggsh.at[pl.ds(sid * RPT, RPT)],
                    out.at[cid, pl.ds(sid * RPT, RPT)])


@functools.cache
def _get_edge_agg():
    mesh = plsc.VectorSubcoreMesh(
        core_axis_name="c", subcore_axis_name="s",
        num_cores=NC, num_subcores=NS)
    return pl.kernel(
        _edge_agg_body,
        out_type=jax.ShapeDtypeStruct((NC, NPAD, F), jnp.float32),
        mesh=mesh,
        scratch_types=[
            pltpu.VMEM((EPP,), jnp.int32),
            pltpu.VMEM((EPP,), jnp.int32),
            pltpu.VMEM((NCH_P, CH), jnp.int32),
            pltpu.VMEM((CH, F), jnp.float32),
            pltpu.VMEM((CH, F), jnp.float32),
            pltpu.VMEM((CH, F), jnp.float32),
            pltpu.VMEM((CH, F), jnp.float32),
            pltpu.VMEM_SHARED((NPAD, F), jnp.float32),
            pltpu.SemaphoreType.DMA,
            pltpu.SemaphoreType.DMA,
            pltpu.SemaphoreType.DMA,
            pltpu.SemaphoreType.DMA,
            pltpu.SemaphoreType.DMA,
        ],
    )


def _edge_agg(htab, src2, et2, dst3):
    return _get_edge_agg()(htab, src2, et2, dst3)


def _head_body(p0_ref, p1_ref, s_ref, Wagg_ref, bagg_ref, Wd1_ref, bd1_ref,
               Wd2_ref, bd2_ref, Wd3_ref, bd3_ref, out_ref, accU):
    i = pl.program_id(0)
    h2 = p0_ref[...] + p1_ref[...] + s_ref[...]
    z = jnp.dot(h2, Wagg_ref[...], preferred_element_type=jnp.float32)
    z = z + bagg_ref[...]
    c = lax.dot_general(z, Wd1_ref[...], (((0,), (0,)), ((), ())),
                        preferred_element_type=jnp.float32)

    @pl.when(i == 0)
    def _init():
        accU[...] = jnp.zeros((8, F), jnp.float32)

    accU[0:1, 0:100] = accU[0:1, 0:100] + c

    @pl.when(i == GRID - 1)
    def _final():
        u = accU[0:1, 0:100] + bd1_ref[...]
        t = jnp.dot(u, Wd2_ref[...], preferred_element_type=jnp.float32)
        t = t + bd2_ref[...]
        t = jnp.where(t > 0, t, 0.01 * t)
        out_ref[...] = (
            jnp.dot(t, Wd3_ref[...], preferred_element_type=jnp.float32)
            + bd3_ref[...]
        )


_head = pl.pallas_call(
    _head_body,
    grid=(GRID,),
    in_specs=[
        pl.BlockSpec((BLK, F), lambda i: (i, 0)),
        pl.BlockSpec((BLK, F), lambda i: (i, 0)),
        pl.BlockSpec((BLK, F), lambda i: (i, 0)),
        pl.BlockSpec((F, 1), lambda i: (0, 0)),
        pl.BlockSpec((1, 1), lambda i: (0, 0)),
        pl.BlockSpec((BLK, 100), lambda i: (i, 0)),
        pl.BlockSpec((1, 100), lambda i: (0, 0)),
        pl.BlockSpec((100, 20), lambda i: (0, 0)),
        pl.BlockSpec((1, 20), lambda i: (0, 0)),
        pl.BlockSpec((20, 10), lambda i: (0, 0)),
        pl.BlockSpec((1, 10), lambda i: (0, 0)),
    ],
    out_specs=pl.BlockSpec((1, 10), lambda i: (0, 0)),
    out_shape=jax.ShapeDtypeStruct((1, 10), jnp.float32),
    scratch_shapes=[
        pltpu.VMEM((8, F), jnp.float32),
    ],
)


def kernel(x, edge_index, etype, V1, comb1, Wself1, b1, V2, comb2, Wself2, b2,
           Wagg, bagg, Wd1, bd1, Wd2, bd2, Wd3, bd3):
    src = edge_index[0].astype(jnp.int32)
    dst = edge_index[1].astype(jnp.int32)
    et = etype.astype(jnp.int32)

    pad = EPAD - E
    src2 = jnp.concatenate([src, jnp.zeros((pad,), jnp.int32)]).reshape(NW, EPT)
    et2 = jnp.concatenate([et, jnp.zeros((pad,), jnp.int32)]).reshape(NW, EPT)
    trash = N + jnp.arange(pad, dtype=jnp.int32) % (NPAD - N)
    dst3 = jnp.concatenate([dst, trash]).reshape(NW, NCHUNK, CH)

    htab1, self1 = _proj_first(x, V1, comb1, Wself1, b1.reshape(1, F))
    parts1 = _edge_agg(htab1.reshape(R * N, F), src2, et2, dst3)

    htab2, self2 = _proj_mid(parts1[0, :N], parts1[1, :N], self1,
                             V2, comb2, Wself2, b2.reshape(1, F))
    parts2 = _edge_agg(htab2.reshape(R * N, F), src2, et2, dst3)

    return _head(parts2[0, :N], parts2[1, :N], self2,
                 Wagg, bagg.reshape(1, 1),
                 Wd1, bd1.reshape(1, 100),
                 Wd2, bd2.reshape(1, 20),
                 Wd3, bd3.reshape(1, 10))


# speedup vs baseline: 17.0009x; 1.0279x over previous
import functools

import jax
import jax.numpy as jnp
from jax import lax
from jax.experimental import pallas as pl
from jax.experimental.pallas import tpu as pltpu
from jax.experimental.pallas import tpu_sc as plsc

N = 10000
E = 320000
F = 128
R = 8
NB = 8

NC = 2
NS = 16
NW = NC * NS

CH = 64
NCHUNK = 320
NBUF = 4
NCH0 = 240
NCH1 = NCHUNK - NCH0
NCP = 40
PH0 = NCH0 // NCP
PH1 = NCH1 // NCP
EPT0 = CH * NCH0
EPT1 = CH * NCH1
EPPC = CH * NCP
C1BASE = NS * EPT0
EPAD = NS * (EPT0 + EPT1)
NPAD = 10240
RPT = NPAD // NS

BLK = 400
GRID = N // BLK


def _proj_body(first, *refs):
    if first:
        x_ref, V_ref, comb_ref, Wself_ref, b_ref, htab_ref, self_ref = refs
        xb = x_ref[...]
    else:
        p0_ref, p1_ref, s_ref, V_ref, comb_ref, Wself_ref, b_ref, \
            htab_ref, self_ref = refs
        xb = p0_ref[...] + p1_ref[...] + s_ref[...]
        xb = jnp.where(xb > 0, xb, 0.01 * xb)

    hbs = [jnp.dot(xb, V_ref[b], preferred_element_type=jnp.float32)
           for b in range(NB)]
    for r in range(R):
        acc = comb_ref[r, 0] * hbs[0]
        for b in range(1, NB):
            acc = acc + comb_ref[r, b] * hbs[b]
        htab_ref[r] = acc

    self_ref[...] = (
        jnp.dot(xb, Wself_ref[...], preferred_element_type=jnp.float32)
        + b_ref[...]
    )


def _make_proj(first):
    node_in = pl.BlockSpec((BLK, F), lambda i: (i, 0))
    in_specs = ([node_in] if first else [node_in, node_in, node_in]) + [
        pl.BlockSpec((NB, F, F), lambda i: (0, 0, 0)),
        pl.BlockSpec((R, NB), lambda i: (0, 0), memory_space=pltpu.SMEM),
        pl.BlockSpec((F, F), lambda i: (0, 0)),
        pl.BlockSpec((1, F), lambda i: (0, 0)),
    ]
    return pl.pallas_call(
        functools.partial(_proj_body, first),
        grid=(GRID,),
        in_specs=in_specs,
        out_specs=[
            pl.BlockSpec((R, BLK, F), lambda i: (0, i, 0)),
            pl.BlockSpec((BLK, F), lambda i: (i, 0)),
        ],
        out_shape=[
            jax.ShapeDtypeStruct((R, N, F), jnp.float32),
            jax.ShapeDtypeStruct((N, F), jnp.float32),
        ],
    )


_proj_first = _make_proj(True)
_proj_mid = _make_proj(False)


def _edge_agg_body(htab, src1, et1, dst2, out, srcv, etv, dstv,
                   r0, r1, r2, r3, aggsh, g0, g1, g2, g3, ssem):
    rows = (r0, r1, r2, r3)
    gsems = (g0, g1, g2, g3)
    cid = lax.axis_index("c")
    sid = lax.axis_index("s")

    def _zrow(i, c):
        for q in range(F // 16):
            r0[i, pl.ds(q * 16, 16)] = jnp.zeros((16,), jnp.float32)
        return c
    lax.fori_loop(0, CH, _zrow, 0)
    for k in range(RPT // CH):
        pltpu.sync_copy(r0, aggsh.at[pl.ds(sid * RPT + k * CH, CH)])

    plsc.subcore_barrier()

    def _run(base, nph):
        ncp, epp = NCP, EPPC
        for p in range(nph):
            eb = base + p * epp
            pltpu.sync_copy(src1.at[pl.ds(eb, epp)], srcv.at[pl.ds(0, epp)])
            pltpu.sync_copy(et1.at[pl.ds(eb, epp)], etv.at[pl.ds(0, epp)])
            pltpu.sync_copy(dst2.at[pl.ds(pl.multiple_of(eb // CH, 8), ncp)],
                            dstv.at[pl.ds(0, ncp)])

            def _gidx(i, c):
                off = pl.multiple_of(i * 16, 16)
                srcv[pl.ds(off, 16)] = (
                    etv[pl.ds(off, 16)] * N + srcv[pl.ds(off, 16)])
                return c
            lax.fori_loop(0, epp // 16, _gidx, 0)

            for q in range(NBUF - 1):
                pltpu.async_copy(
                    htab.at[srcv.at[pl.ds(q * CH, CH)]], rows[q], gsems[q])

            def _ring(jj, c):
                for q in range(NBUF):
                    j = jj * NBUF + q
                    off = pl.multiple_of(j * CH, CH)
                    pltpu.make_async_copy(
                        htab.at[srcv.at[pl.ds(off, CH)]], rows[q],
                        gsems[q]).wait()
                    pltpu.async_copy(rows[q], aggsh.at[dstv.at[j]], ssem,
                                     add=True)
                    if p == 0 and q == 0:
                        @pl.when(jj > 0)
                        def _drain0():
                            pltpu.make_async_copy(
                                rows[q], aggsh.at[dstv.at[j]], ssem).wait()
                    else:
                        pltpu.make_async_copy(
                            rows[q], aggsh.at[dstv.at[j]], ssem).wait()

                    @pl.when(j + NBUF - 1 < ncp)
                    def _refill():
                        offn = pl.multiple_of((j + NBUF - 1) * CH, CH)
                        pltpu.async_copy(
                            htab.at[srcv.at[pl.ds(offn, CH)]],
                            rows[(q + NBUF - 1) % NBUF],
                            gsems[(q + NBUF - 1) % NBUF])
                return c
            lax.fori_loop(0, ncp // NBUF, _ring, 0)

        pltpu.make_async_copy(r0, aggsh.at[dstv.at[0]], ssem).wait()

    @pl.when(cid == 0)
    def _core0():
        _run(sid * EPT0, PH0)

    @pl.when(cid == 1)
    def _core1():
        _run(C1BASE + sid * EPT1, PH1)

    plsc.subcore_barrier()

    pltpu.sync_copy(aggsh.at[pl.ds(sid * RPT, RPT)],
                    out.at[cid, pl.ds(sid * RPT, RPT)])


@functools.cache
def _get_edge_agg():
    mesh = plsc.VectorSubcoreMesh(
        core_axis_name="c", subcore_axis_name="s",
        num_cores=NC, num_subcores=NS)
    return pl.kernel(
        _edge_agg_body,
        out_type=jax.ShapeDtypeStruct((NC, NPAD, F), jnp.float32),
        mesh=mesh,
        scratch_types=[
            pltpu.VMEM((EPPC,), jnp.int32),
            pltpu.VMEM((EPPC,), jnp.int32),
            pltpu.VMEM((NCP, CH), jnp.int32),
            pltpu.VMEM((CH, F), jnp.float32),
            pltpu.VMEM((CH, F), jnp.float32),
            pltpu.VMEM((CH, F), jnp.float32),
            pltpu.VMEM((CH, F), jnp.float32),
            pltpu.VMEM_SHARED((NPAD, F), jnp.float32),
            pltpu.SemaphoreType.DMA,
            pltpu.SemaphoreType.DMA,
            pltpu.SemaphoreType.DMA,
            pltpu.SemaphoreType.DMA,
            pltpu.SemaphoreType.DMA,
        ],
    )


def _edge_agg(htab, src2, et2, dst3):
    return _get_edge_agg()(htab, src2, et2, dst3)


def _head_body(p0_ref, p1_ref, s_ref, Wagg_ref, bagg_ref, Wd1_ref, bd1_ref,
               Wd2_ref, bd2_ref, Wd3_ref, bd3_ref, out_ref, accU):
    i = pl.program_id(0)
    h2 = p0_ref[...] + p1_ref[...] + s_ref[...]
    z = jnp.dot(h2, Wagg_ref[...], preferred_element_type=jnp.float32)
    z = z + bagg_ref[...]
    c = lax.dot_general(z, Wd1_ref[...], (((0,), (0,)), ((), ())),
                        preferred_element_type=jnp.float32)

    @pl.when(i == 0)
    def _init():
        accU[...] = jnp.zeros((8, F), jnp.float32)

    accU[0:1, 0:100] = accU[0:1, 0:100] + c

    @pl.when(i == GRID - 1)
    def _final():
        u = accU[0:1, 0:100] + bd1_ref[...]
        t = jnp.dot(u, Wd2_ref[...], preferred_element_type=jnp.float32)
        t = t + bd2_ref[...]
        t = jnp.where(t > 0, t, 0.01 * t)
        out_ref[...] = (
            jnp.dot(t, Wd3_ref[...], preferred_element_type=jnp.float32)
            + bd3_ref[...]
        )


_head = pl.pallas_call(
    _head_body,
    grid=(GRID,),
    in_specs=[
        pl.BlockSpec((BLK, F), lambda i: (i, 0)),
        pl.BlockSpec((BLK, F), lambda i: (i, 0)),
        pl.BlockSpec((BLK, F), lambda i: (i, 0)),
        pl.BlockSpec((F, 1), lambda i: (0, 0)),
        pl.BlockSpec((1, 1), lambda i: (0, 0)),
        pl.BlockSpec((BLK, 100), lambda i: (i, 0)),
        pl.BlockSpec((1, 100), lambda i: (0, 0)),
        pl.BlockSpec((100, 20), lambda i: (0, 0)),
        pl.BlockSpec((1, 20), lambda i: (0, 0)),
        pl.BlockSpec((20, 10), lambda i: (0, 0)),
        pl.BlockSpec((1, 10), lambda i: (0, 0)),
    ],
    out_specs=pl.BlockSpec((1, 10), lambda i: (0, 0)),
    out_shape=jax.ShapeDtypeStruct((1, 10), jnp.float32),
    scratch_shapes=[
        pltpu.VMEM((8, F), jnp.float32),
    ],
)


def kernel(x, edge_index, etype, V1, comb1, Wself1, b1, V2, comb2, Wself2, b2,
           Wagg, bagg, Wd1, bd1, Wd2, bd2, Wd3, bd3):
    src = edge_index[0].astype(jnp.int32)
    dst = edge_index[1].astype(jnp.int32)
    et = etype.astype(jnp.int32)

    pad = EPAD - E
    src1 = jnp.concatenate([src, jnp.zeros((pad,), jnp.int32)])
    et1 = jnp.concatenate([et, jnp.zeros((pad,), jnp.int32)])
    trash = N + jnp.arange(pad, dtype=jnp.int32) % (NPAD - N)
    dst2 = jnp.concatenate([dst, trash]).reshape(EPAD // CH, CH)

    htab1, self1 = _proj_first(x, V1, comb1, Wself1, b1.reshape(1, F))
    parts1 = _edge_agg(htab1.reshape(R * N, F), src1, et1, dst2)

    htab2, self2 = _proj_mid(parts1[0, :N], parts1[1, :N], self1,
                             V2, comb2, Wself2, b2.reshape(1, F))
    parts2 = _edge_agg(htab2.reshape(R * N, F), src1, et1, dst2)

    return _head(parts2[0, :N], parts2[1, :N], self2,
                 Wagg, bagg.reshape(1, 1),
                 Wd1, bd1.reshape(1, 100),
                 Wd2, bd2.reshape(1, 20),
                 Wd3, bd3.reshape(1, 10))
